# baseline (speedup 1.0000x reference)
import functools

import numpy as np
import jax
import jax.numpy as jnp
from jax import lax
from jax.experimental import pallas as pl
from jax.experimental.pallas import tpu as pltpu
from jax.experimental.pallas import tpu_sc as plsc

_VAR = np.array([(0, 1)] * 10 + [(0, 2)] * 15, dtype=np.int32)
_MEM = np.array(
    [(1, 0), (1, 1), (1, 2), (1, 3), (1, 4), (3, 4), (3, 3), (3, 2), (3, 1),
     (3, 0), (1, 0), (1, 1), (1, 2), (1, 3), (1, 4), (2, 0), (2, 1), (2, 2),
     (2, 3), (2, 4), (3, 0), (3, 1), (3, 2), (3, 3), (3, 4)], dtype=np.int32)
_FLAT = _VAR * 5 + _MEM
_FA = tuple(int(v) for v in _FLAT[:, 0])
_FB = tuple(int(v) for v in _FLAT[:, 1])
_USED = tuple(sorted(set(_FA) | set(_FB)))

_N = 1_000_000
_NF = 15
_NR = 25
_CH = 800
_NCHUNK = _N // _CH
_NW = 32
_GROUPS = _CH // 16
_KMAX = -(-_NCHUNK // _NW)

def _sc_body(x_hbm, o_hbm, xbuf, obuf):
    wid = lax.axis_index("s") * 2 + lax.axis_index("c")
    lanes = lax.iota(jnp.int32, 16)
    splats = {i: jnp.full((16,), i, jnp.int32) for i in range(_NR)}

    def chunk_step(k, carry):
        chunk = k * _NW + wid

        @pl.when(chunk < _NCHUNK)
        def _():
            cb = pl.multiple_of(chunk * _CH, 8)
            pltpu.sync_copy(x_hbm.at[pl.ds(cb, _CH)], xbuf)

            def group_step(g, gcarry):
                cases = g * 16 + lanes
                feats = {}
                for f in _USED:
                    v, m = divmod(f, 5)
                    feats[f] = plsc.load_gather(
                        xbuf, [cases, splats[v], splats[m]])
                for r in range(_NR):
                    v = jnp.minimum(feats[_FA[r]], feats[_FB[r]])
                    plsc.store_scatter(obuf, [cases, splats[r]], v)
                return gcarry

            lax.fori_loop(0, _GROUPS, group_step, 0)
            pltpu.sync_copy(obuf, o_hbm.at[pl.ds(cb, _CH)])

        return carry

    lax.fori_loop(0, _KMAX, chunk_step, 0)


@functools.cache
def _sc_run():
    mesh = plsc.VectorSubcoreMesh(
        core_axis_name="c", subcore_axis_name="s",
        num_cores=2, num_subcores=16)
    return pl.kernel(
        _sc_body,
        out_type=jax.ShapeDtypeStruct((_N, _NR), jnp.float32),
        mesh=mesh,
        compiler_params=pltpu.CompilerParams(
            needs_layout_passes=False, use_tc_tiling_on_sc=False),
        scratch_types=[
            pltpu.VMEM((_CH, 3, 5), jnp.float32),
            pltpu.VMEM((_CH, _NR), jnp.float32),
        ],
    )


@jax.jit
def kernel(x):
    return _sc_run()(x)


# speedup vs baseline: 13.4330x; 13.4330x over previous
import functools

import numpy as np
import jax
import jax.numpy as jnp
from jax import lax
from jax.experimental import pallas as pl
from jax.experimental.pallas import tpu as pltpu
from jax.experimental.pallas import tpu_sc as plsc

_VAR = np.array([(0, 1)] * 10 + [(0, 2)] * 15, dtype=np.int32)
_MEM = np.array(
    [(1, 0), (1, 1), (1, 2), (1, 3), (1, 4), (3, 4), (3, 3), (3, 2), (3, 1),
     (3, 0), (1, 0), (1, 1), (1, 2), (1, 3), (1, 4), (2, 0), (2, 1), (2, 2),
     (2, 3), (2, 4), (3, 0), (3, 1), (3, 2), (3, 3), (3, 4)], dtype=np.int32)
_FLAT = _VAR * 5 + _MEM
_FA = tuple(int(v) for v in _FLAT[:, 0])
_FB = tuple(int(v) for v in _FLAT[:, 1])
_FEATS = tuple(sorted(set(_FA) | set(_FB)))
_FIDX = {f: i for i, f in enumerate(_FEATS)}
_NFEAT = len(_FEATS)

_N = 1_000_000
_NR = 25
_NRP = 32
_CB = 512
_NBLK = (_N - 64) // _CB
_TAIL = _NBLK * _CB
_NW = 32
_KMAX = -(-_NBLK // _NW)

_NC = _N // _CB


def _sc_body(xt_hbm, ot_hbm, xbuf, obuf):
    wid = lax.axis_index("s") * 2 + lax.axis_index("c")

    def compute(nc):
        def group(g, gcarry):
            o = g * 16
            vals = {f: xbuf[f % 5, f // 5, pl.ds(o, 16)] for f in _FEATS}
            for r in range(_NR):
                w = jnp.minimum(vals[_FA[r]], vals[_FB[r]])
                obuf[r // 8, r % 8, pl.ds(o, 16)] = w
            return gcarry

        lax.fori_loop(0, nc // 16, group, 0)

    def chunk_step(k, carry):
        b = k * _NW + wid

        @pl.when(b < _NBLK)
        def _():
            cb = pl.multiple_of(b * _CB, 128)
            for m in range(5):
                pltpu.sync_copy(xt_hbm.at[m, :, pl.ds(cb, _CB)], xbuf.at[m])
            compute(_CB)
            for rt in range(4):
                pltpu.sync_copy(obuf.at[rt],
                                ot_hbm.at[pl.ds(rt * 8, 8), pl.ds(cb, _CB)])

        return carry

    lax.fori_loop(0, _KMAX, chunk_step, 0)


@functools.cache
def _sc_run():
    mesh = plsc.VectorSubcoreMesh(
        core_axis_name="c", subcore_axis_name="s",
        num_cores=2, num_subcores=16)
    return pl.kernel(
        _sc_body,
        out_type=jax.ShapeDtypeStruct((_NRP, _N), jnp.float32),
        mesh=mesh,
        compiler_params=pltpu.CompilerParams(needs_layout_passes=False),
        scratch_types=[
            pltpu.VMEM((5, 3, _CB), jnp.float32),
            pltpu.VMEM((4, 8, _CB), jnp.float32),
        ],
    )


@jax.jit
def kernel(x):
    xt = jnp.transpose(x, (2, 1, 0))
    ot = _sc_run()(xt)
    xtail = lax.slice(x, (_TAIL, 0, 0), (_N, 3, 5))
    wtail = jnp.min(xtail[:, _VAR, _MEM], axis=2)
    wt32 = jnp.pad(wtail.T, ((0, _NRP - _NR), (0, 0)))
    ot = lax.dynamic_update_slice(ot, wt32, (0, _TAIL))
    return jnp.transpose(ot, (1, 0))[:, :_NR]


# speedup vs baseline: 24.4324x; 1.8188x over previous
import functools

import numpy as np
import jax
import jax.numpy as jnp
from jax import lax
from jax.experimental import pallas as pl
from jax.experimental.pallas import tpu as pltpu
from jax.experimental.pallas import tpu_sc as plsc

_VAR = np.array([(0, 1)] * 10 + [(0, 2)] * 15, dtype=np.int32)
_MEM = np.array(
    [(1, 0), (1, 1), (1, 2), (1, 3), (1, 4), (3, 4), (3, 3), (3, 2), (3, 1),
     (3, 0), (1, 0), (1, 1), (1, 2), (1, 3), (1, 4), (2, 0), (2, 1), (2, 2),
     (2, 3), (2, 4), (3, 0), (3, 1), (3, 2), (3, 3), (3, 4)], dtype=np.int32)
_FLAT = _VAR * 5 + _MEM
_FA = tuple(int(v) for v in _FLAT[:, 0])
_FB = tuple(int(v) for v in _FLAT[:, 1])
_FEATS = tuple(sorted(set(_FA) | set(_FB)))
_FIDX = {f: i for i, f in enumerate(_FEATS)}
_NFEAT = len(_FEATS)

_N = 1_000_000
_NR = 25
_NRP = 32
_CB = 512
_NBLK = (_N - 64) // _CB
_TAIL = _NBLK * _CB
_NW = 32
_KMAX = -(-_NBLK // _NW)

_NC = _N // _CB


def _sc_body(xt_hbm, ot_hbm, xbuf, obuf):
    wid = lax.axis_index("s") * 2 + lax.axis_index("c")

    def compute(nc):
        def group(g, gcarry):
            o = g * 16
            vals = {f: xbuf[f % 5, f // 5, pl.ds(o, 16)] for f in _FEATS}
            for r in range(_NR):
                w = jnp.minimum(vals[_FA[r]], vals[_FB[r]])
                obuf[r, pl.ds(o, 16)] = w
            return gcarry

        lax.fori_loop(0, nc // 16, group, 0)

    def chunk_step(k, carry):
        b = k * _NW + wid

        @pl.when(b < _NBLK)
        def _():
            cb = pl.multiple_of(b * _CB, 128)
            pltpu.sync_copy(xt_hbm.at[:, :, pl.ds(cb, _CB)], xbuf)
            compute(_CB)
            pltpu.sync_copy(obuf, ot_hbm.at[:, pl.ds(cb, _CB)])

        return carry

    lax.fori_loop(0, _KMAX, chunk_step, 0)


@functools.cache
def _sc_run():
    mesh = plsc.VectorSubcoreMesh(
        core_axis_name="c", subcore_axis_name="s",
        num_cores=2, num_subcores=16)
    return pl.kernel(
        _sc_body,
        out_type=jax.ShapeDtypeStruct((_NRP, _N), jnp.float32),
        mesh=mesh,
        compiler_params=pltpu.CompilerParams(needs_layout_passes=False),
        scratch_types=[
            pltpu.VMEM((5, 3, _CB), jnp.float32),
            pltpu.VMEM((_NRP, _CB), jnp.float32),
        ],
    )


@jax.jit
def kernel(x):
    xt = jnp.transpose(x, (2, 1, 0))
    ot = _sc_run()(xt)
    xtail = lax.slice(x, (_TAIL, 0, 0), (_N, 3, 5))
    wtail = jnp.min(xtail[:, _VAR, _MEM], axis=2)
    wt32 = jnp.pad(wtail.T, ((0, _NRP - _NR), (0, 0)))
    ot = lax.dynamic_update_slice(ot, wt32, (0, _TAIL))
    return jnp.transpose(ot, (1, 0))[:, :_NR]


# speedup vs baseline: 44.6675x; 1.8282x over previous
import functools

import numpy as np
import jax
import jax.numpy as jnp
from jax import lax
from jax.experimental import pallas as pl
from jax.experimental.pallas import tpu as pltpu
from jax.experimental.pallas import tpu_sc as plsc

_VAR = np.array([(0, 1)] * 10 + [(0, 2)] * 15, dtype=np.int32)
_MEM = np.array(
    [(1, 0), (1, 1), (1, 2), (1, 3), (1, 4), (3, 4), (3, 3), (3, 2), (3, 1),
     (3, 0), (1, 0), (1, 1), (1, 2), (1, 3), (1, 4), (2, 0), (2, 1), (2, 2),
     (2, 3), (2, 4), (3, 0), (3, 1), (3, 2), (3, 3), (3, 4)], dtype=np.int32)
_FLAT = _VAR * 5 + _MEM
_FA = tuple(int(v) for v in _FLAT[:, 0])
_FB = tuple(int(v) for v in _FLAT[:, 1])
_FEATS = tuple(sorted(set(_FA) | set(_FB)))
_FIDX = {f: i for i, f in enumerate(_FEATS)}
_NFEAT = len(_FEATS)

_N = 1_000_000
_NR = 25
_NRP = 32
_CB = 512
_NBLK = (_N - 64) // _CB
_TAIL = _NBLK * _CB
_NW = 32
_KMAX = -(-_NBLK // _NW)

_NC = _N // _CB


def _sc_body(xt_hbm, ot_hbm, xbuf, obuf, si0, si1, so0, so1):
    wid = lax.axis_index("s") * 2 + lax.axis_index("c")
    sin = (si0, si1)
    sout = (so0, so1)

    def cbase(k):
        return pl.multiple_of((k * _NW + wid) * _CB, 128)

    def start_in(k, p):
        @pl.when(k * _NW + wid < _NBLK)
        def _():
            pltpu.async_copy(xt_hbm.at[:, :, pl.ds(cbase(k), _CB)],
                             xbuf.at[p], sin[p])

    def compute(p):
        def group(g, gcarry):
            o = g * 16
            vals = {f: xbuf[p, f % 5, f // 5, pl.ds(o, 16)] for f in _FEATS}
            for r in range(_NR):
                w = jnp.minimum(vals[_FA[r]], vals[_FB[r]])
                obuf[p, r, pl.ds(o, 16)] = w
            return gcarry

        lax.fori_loop(0, _CB // 16, group, 0)

    def phase(k, p):
        b = k * _NW + wid
        cb = cbase(k)

        @pl.when(b < _NBLK)
        def _():
            pltpu.make_async_copy(xt_hbm.at[:, :, pl.ds(cb, _CB)],
                                  xbuf.at[p], sin[p]).wait()

            @pl.when(b >= 2 * _NW)
            def _():
                pltpu.make_async_copy(obuf.at[p], ot_hbm.at[:, pl.ds(cb, _CB)],
                                      sout[p]).wait()

            compute(p)
            pltpu.async_copy(obuf.at[p], ot_hbm.at[:, pl.ds(cb, _CB)], sout[p])

        start_in(k + 2, p)

    start_in(0, 0)
    start_in(1, 1)

    def j_body(j, carry):
        phase(2 * j, 0)
        phase(2 * j + 1, 1)
        return carry

    lax.fori_loop(0, _KMAX // 2, j_body, 0)

    for p in range(2):
        @pl.when(p * _NW + wid < _NBLK)
        def _(p=p):
            pltpu.make_async_copy(obuf.at[p], ot_hbm.at[:, pl.ds(0, _CB)],
                                  sout[p]).wait()


@functools.cache
def _sc_run():
    mesh = plsc.VectorSubcoreMesh(
        core_axis_name="c", subcore_axis_name="s",
        num_cores=2, num_subcores=16)
    return pl.kernel(
        _sc_body,
        out_type=jax.ShapeDtypeStruct((_NRP, _N), jnp.float32),
        mesh=mesh,
        compiler_params=pltpu.CompilerParams(needs_layout_passes=False),
        scratch_types=[
            pltpu.VMEM((2, 5, 3, _CB), jnp.float32),
            pltpu.VMEM((2, _NRP, _CB), jnp.float32),
            pltpu.SemaphoreType.DMA,
            pltpu.SemaphoreType.DMA,
            pltpu.SemaphoreType.DMA,
            pltpu.SemaphoreType.DMA,
        ],
    )


@jax.jit
def kernel(x):
    xt = jnp.transpose(x, (2, 1, 0))
    ot = _sc_run()(xt)
    xtail = lax.slice(x, (_TAIL, 0, 0), (_N, 3, 5))
    wtail = jnp.min(xtail[:, _VAR, _MEM], axis=2)
    wt32 = jnp.pad(wtail.T, ((0, _NRP - _NR), (0, 0)))
    ot = lax.dynamic_update_slice(ot, wt32, (0, _TAIL))
    return jnp.transpose(ot, (1, 0))[:, :_NR]


# speedup vs baseline: 45.0193x; 1.0079x over previous
import functools

import numpy as np
import jax
import jax.numpy as jnp
from jax import lax
from jax.experimental import pallas as pl
from jax.experimental.pallas import tpu as pltpu
from jax.experimental.pallas import tpu_sc as plsc

_VAR = np.array([(0, 1)] * 10 + [(0, 2)] * 15, dtype=np.int32)
_MEM = np.array(
    [(1, 0), (1, 1), (1, 2), (1, 3), (1, 4), (3, 4), (3, 3), (3, 2), (3, 1),
     (3, 0), (1, 0), (1, 1), (1, 2), (1, 3), (1, 4), (2, 0), (2, 1), (2, 2),
     (2, 3), (2, 4), (3, 0), (3, 1), (3, 2), (3, 3), (3, 4)], dtype=np.int32)
_FLAT = _VAR * 5 + _MEM
_FA = tuple(int(v) for v in _FLAT[:, 0])
_FB = tuple(int(v) for v in _FLAT[:, 1])
_FEATS = tuple(sorted(set(_FA) | set(_FB)))
_FIDX = {f: i for i, f in enumerate(_FEATS)}
_NFEAT = len(_FEATS)

_N = 1_000_000
_NR = 25
_NRP = 32
_CB = 1024
_NBLK = 976
_TAIL = _NBLK * _CB
_NW = 32
_KMAX = 32

_NC = _N // _CB


def _sc_body(xt_hbm, ot_hbm, xbuf, obuf, si0, si1, so0, so1):
    wid = lax.axis_index("s") * 2 + lax.axis_index("c")
    sin = (si0, si1)
    sout = (so0, so1)

    def cbase(k):
        return pl.multiple_of((k * _NW + wid) * _CB, 128)

    def start_in(k, p):
        @pl.when(k * _NW + wid < _NBLK)
        def _():
            pltpu.async_copy(xt_hbm.at[:, :, pl.ds(cbase(k), _CB)],
                             xbuf.at[p], sin[p])

    def compute(p):
        def group(g, gcarry):
            o = g * 16
            vals = {f: xbuf[p, f % 5, f // 5, pl.ds(o, 16)] for f in _FEATS}
            for r in range(_NR):
                w = jnp.minimum(vals[_FA[r]], vals[_FB[r]])
                obuf[p, r, pl.ds(o, 16)] = w
            return gcarry

        lax.fori_loop(0, _CB // 16, group, 0)

    def phase(k, p):
        b = k * _NW + wid
        cb = cbase(k)

        @pl.when(b < _NBLK)
        def _():
            pltpu.make_async_copy(xt_hbm.at[:, :, pl.ds(cb, _CB)],
                                  xbuf.at[p], sin[p]).wait()

            @pl.when(b >= 2 * _NW)
            def _():
                pltpu.make_async_copy(obuf.at[p], ot_hbm.at[:, pl.ds(cb, _CB)],
                                      sout[p]).wait()

            compute(p)
            pltpu.async_copy(obuf.at[p], ot_hbm.at[:, pl.ds(cb, _CB)], sout[p])

        start_in(k + 2, p)

    start_in(0, 0)
    start_in(1, 1)

    def j_body(j, carry):
        phase(2 * j, 0)
        phase(2 * j + 1, 1)
        return carry

    lax.fori_loop(0, _KMAX // 2, j_body, 0)

    for p in range(2):
        @pl.when(p * _NW + wid < _NBLK)
        def _(p=p):
            pltpu.make_async_copy(obuf.at[p], ot_hbm.at[:, pl.ds(0, _CB)],
                                  sout[p]).wait()


@functools.cache
def _sc_run():
    mesh = plsc.VectorSubcoreMesh(
        core_axis_name="c", subcore_axis_name="s",
        num_cores=2, num_subcores=16)
    return pl.kernel(
        _sc_body,
        out_type=jax.ShapeDtypeStruct((_NRP, _N), jnp.float32),
        mesh=mesh,
        compiler_params=pltpu.CompilerParams(needs_layout_passes=False),
        scratch_types=[
            pltpu.VMEM((2, 5, 3, _CB), jnp.float32),
            pltpu.VMEM((2, _NRP, _CB), jnp.float32),
            pltpu.SemaphoreType.DMA,
            pltpu.SemaphoreType.DMA,
            pltpu.SemaphoreType.DMA,
            pltpu.SemaphoreType.DMA,
        ],
    )


@jax.jit
def kernel(x):
    xt = jnp.transpose(x, (2, 1, 0))
    ot = _sc_run()(xt)
    xtail = lax.slice(x, (_TAIL, 0, 0), (_N, 3, 5))
    wtail = jnp.min(xtail[:, _VAR, _MEM], axis=2)
    wt32 = jnp.pad(wtail.T, ((0, _NRP - _NR), (0, 0)))
    ot = lax.dynamic_update_slice(ot, wt32, (0, _TAIL))
    return jnp.transpose(ot, (1, 0))[:, :_NR]
